# Initial kernel scaffold; baseline (speedup 1.0000x reference)
import functools

import jax
import jax.numpy as jnp
from jax import lax
from jax.experimental import pallas as pl
from jax.experimental.pallas import tpu as pltpu
from jax.experimental.pallas import tpu_sc as plsc

D = 128
CB = 4
CW = D // CB
K = 120
NC = 2
NS = 16
DEGW = 8


@functools.lru_cache(maxsize=None)
def _sc_call(NU, NI, E):
    N = max(NU, NI)
    NPAD = -(-(N + 8) // (NS * 8)) * (NS * 8)
    RPT = NPAD // NS
    EP = -(-E // K) * K
    NCH = EP // K
    ITERS = -(-NCH // NS)
    GUARD = (NCH % NS) != 0

    mesh = plsc.VectorSubcoreMesh(core_axis_name="c", subcore_axis_name="s")
    f32 = jnp.float32

    def body(fu0, fu1, fu2, fu3, fi0, fi1, fi2, fi3,
             gi_i, si_i, gi_u, si_u, ones_h, z32_h, z8_h,
             oi0, oi1, oi2, oi3, ou0, ou1, ou2, ou3, di8, du8,
             acc, dega, gidx, sidx, rows, ones_v, sem):
        c = lax.axis_index("c")
        s = lax.axis_index("s")
        pltpu.sync_copy(ones_h, ones_v)

        def run_side(tables, gi, si, outs, dego):
            for cb in range(CB):
                pltpu.sync_copy(z32_h, acc.at[pl.ds(s * RPT, RPT)])
                if cb == 0:
                    pltpu.sync_copy(z8_h, dega.at[pl.ds(s * RPT, RPT)])
                plsc.subcore_barrier()

                table = tables[cb]

                def chunk_body(j, carry, *, _cb=cb, _table=table, _gi=gi, _si=si):
                    ch = j * NS + s

                    def do_chunk():
                        base = ch * K
                        pltpu.sync_copy(_gi.at[pl.ds(base, K)], gidx)
                        pltpu.sync_copy(_si.at[pl.ds(base, K)], sidx)
                        pltpu.async_copy(_table.at[gidx], rows, sem).wait()
                        pltpu.sync_copy(rows, acc.at[sidx], add=True)
                        if _cb == 0:
                            pltpu.sync_copy(ones_v, dega.at[sidx], add=True)

                    if GUARD:
                        pl.when(ch < NCH)(do_chunk)
                    else:
                        do_chunk()
                    return carry

                lax.fori_loop(0, ITERS, chunk_body, jnp.int32(0))
                plsc.subcore_barrier()
                pltpu.sync_copy(acc.at[pl.ds(s * RPT, RPT)],
                                outs[cb].at[pl.ds(s * RPT, RPT)])
                if cb == 0:
                    pltpu.sync_copy(dega.at[pl.ds(s * RPT, RPT)],
                                    dego.at[pl.ds(s * RPT, RPT)])
                plsc.subcore_barrier()

        @pl.when(c == 0)
        def _():
            run_side([fu0, fu1, fu2, fu3], gi_i, si_i, [oi0, oi1, oi2, oi3], di8)

        @pl.when(c == 1)
        def _():
            run_side([fi0, fi1, fi2, fi3], gi_u, si_u, [ou0, ou1, ou2, ou3], du8)

    out_type = ([jax.ShapeDtypeStruct((NPAD, CW), f32) for _ in range(2 * CB)]
                + [jax.ShapeDtypeStruct((NPAD, DEGW), f32) for _ in range(2)])
    scratch_types = [
        pltpu.VMEM_SHARED((NPAD, CW), f32),
        pltpu.VMEM_SHARED((NPAD, DEGW), f32),
        pltpu.VMEM((K,), jnp.int32),
        pltpu.VMEM((K,), jnp.int32),
        pltpu.VMEM((K, CW), f32),
        pltpu.VMEM((K, DEGW), f32),
        pltpu.SemaphoreType.DMA,
    ]
    return pl.kernel(body, out_type=out_type, mesh=mesh,
                     scratch_types=scratch_types)


def _mean_body(b0, b1, b2, b3, dg, out):
    d = jnp.maximum(dg[:, 0:1], 1.0)
    out[:, 0 * CW:1 * CW] = b0[...] / d
    out[:, 1 * CW:2 * CW] = b1[...] / d
    out[:, 2 * CW:3 * CW] = b2[...] / d
    out[:, 3 * CW:4 * CW] = b3[...] / d


def _mean(blocks, deg8, n):
    bs = 2000
    grid = (pl.cdiv(n, bs),)
    return pl.pallas_call(
        _mean_body,
        grid=grid,
        in_specs=[pl.BlockSpec((bs, CW), lambda i: (i, 0)) for _ in range(CB)]
        + [pl.BlockSpec((bs, DEGW), lambda i: (i, 0))],
        out_specs=pl.BlockSpec((bs, D), lambda i: (i, 0)),
        out_shape=jax.ShapeDtypeStruct((n, D), jnp.float32),
    )(*blocks, deg8)


def kernel(feat_user, feat_item, edge_index):
    NU, _ = feat_user.shape
    NI, _ = feat_item.shape
    E = edge_index.shape[1]
    N = max(NU, NI)

    src = edge_index[0].astype(jnp.int32)
    dst = edge_index[1].astype(jnp.int32)
    EP = -(-E // K) * K
    if EP != E:
        pad_g = jnp.zeros((EP - E,), jnp.int32)
        pad_s = jnp.full((EP - E,), N, jnp.int32)
        gsrc = jnp.concatenate([src, pad_g])
        ssrc = jnp.concatenate([src, pad_s])
        gdst = jnp.concatenate([dst, pad_g])
        sdst = jnp.concatenate([dst, pad_s])
    else:
        gsrc, ssrc, gdst, sdst = src, src, dst, dst

    fu = [feat_user[:, cb * CW:(cb + 1) * CW] for cb in range(CB)]
    fi = [feat_item[:, cb * CW:(cb + 1) * CW] for cb in range(CB)]

    call = _sc_call(NU, NI, E)
    NPAD = -(-(N + 8) // (NS * 8)) * (NS * 8)
    RPT = NPAD // NS
    ones_h = jnp.ones((K, DEGW), jnp.float32)
    z32_h = jnp.zeros((RPT, CW), jnp.float32)
    z8_h = jnp.zeros((RPT, DEGW), jnp.float32)

    outs = call(*fu, *fi, gsrc, sdst, gdst, ssrc, ones_h, z32_h, z8_h)
    si_blocks = outs[0:4]
    su_blocks = outs[4:8]
    di8, du8 = outs[8], outs[9]

    h_item = _mean(si_blocks, di8, NI)
    h_user = _mean(su_blocks, du8, NU)
    return (h_user, h_item)


# speedup vs baseline: 2.7863x; 2.7863x over previous
import functools

import jax
import jax.numpy as jnp
from jax import lax
from jax.experimental import pallas as pl
from jax.experimental.pallas import tpu as pltpu
from jax.experimental.pallas import tpu_sc as plsc

D = 128
CB = 4
CW = D // CB
K = 120
NC = 2
NS = 16
DEGW = 8


@functools.lru_cache(maxsize=None)
def _sc_call(NU, NI, E):
    N = max(NU, NI)
    NPAD = -(-(N + 8) // (NS * 8)) * (NS * 8)
    RPT = NPAD // NS
    EP = -(-E // K) * K
    NCH = EP // K
    ITERS = -(-NCH // NS)
    GUARD = (NCH % NS) != 0

    mesh = plsc.VectorSubcoreMesh(core_axis_name="c", subcore_axis_name="s")
    f32 = jnp.float32

    def body(fu0, fu1, fu2, fu3, fi0, fi1, fi2, fi3,
             gi_i, si_i, gi_u, si_u, ones_h, z32_h, z8_h,
             oi0, oi1, oi2, oi3, ou0, ou1, ou2, ou3, di8, du8,
             acc, dega, gidx, sidx, rows, ones_v, sem):
        c = lax.axis_index("c")
        s = lax.axis_index("s")
        pltpu.sync_copy(ones_h, ones_v)

        def run_side(tables, gi, si, outs, dego):
            for cb in range(CB):
                pltpu.sync_copy(z32_h, acc.at[pl.ds(s * RPT, RPT)])
                if cb == 0:
                    pltpu.sync_copy(z8_h, dega.at[pl.ds(s * RPT, RPT)])
                plsc.subcore_barrier()

                table = tables[cb]

                def chunk_body(j, carry, *, _cb=cb, _table=table, _gi=gi, _si=si):
                    ch = j * NS + s

                    def do_chunk():
                        base = ch * K
                        pltpu.sync_copy(_gi.at[pl.ds(base, K)], gidx)
                        pltpu.sync_copy(_si.at[pl.ds(base, K)], sidx)
                        pltpu.async_copy(_table.at[gidx], rows, sem).wait()
                        pltpu.sync_copy(rows, acc.at[sidx], add=True)
                        if _cb == 0:
                            pltpu.sync_copy(ones_v, dega.at[sidx], add=True)

                    if GUARD:
                        pl.when(ch < NCH)(do_chunk)
                    else:
                        do_chunk()
                    return carry

                lax.fori_loop(0, ITERS, chunk_body, jnp.int32(0))
                plsc.subcore_barrier()
                pltpu.sync_copy(acc.at[pl.ds(s * RPT, RPT)],
                                outs[cb].at[pl.ds(s * RPT, RPT)])
                if cb == 0:
                    pltpu.sync_copy(dega.at[pl.ds(s * RPT, RPT)],
                                    dego.at[pl.ds(s * RPT, RPT)])
                plsc.subcore_barrier()

        @pl.when(c == 0)
        def _():
            run_side([fu0, fu1, fu2, fu3], gi_i, si_i, [oi0, oi1, oi2, oi3], di8)

        @pl.when(c == 1)
        def _():
            run_side([fi0, fi1, fi2, fi3], gi_u, si_u, [ou0, ou1, ou2, ou3], du8)

    out_type = ([jax.ShapeDtypeStruct((NPAD, CW), f32) for _ in range(2 * CB)]
                + [jax.ShapeDtypeStruct((NPAD, DEGW), f32) for _ in range(2)])
    scratch_types = [
        pltpu.VMEM_SHARED((NPAD, CW), f32),
        pltpu.VMEM_SHARED((NPAD, DEGW), f32),
        pltpu.VMEM((K,), jnp.int32),
        pltpu.VMEM((K,), jnp.int32),
        pltpu.VMEM((K, CW), f32),
        pltpu.VMEM((K, DEGW), f32),
        pltpu.SemaphoreType.DMA,
    ]
    return pl.kernel(body, out_type=out_type, mesh=mesh,
                     scratch_types=scratch_types,
                     compiler_params=pltpu.CompilerParams(
                         use_tc_tiling_on_sc=False))


def _mean_body(b0, b1, b2, b3, dg, out):
    d = jnp.maximum(dg[:, 0:1], 1.0)
    out[:, 0 * CW:1 * CW] = b0[...] / d
    out[:, 1 * CW:2 * CW] = b1[...] / d
    out[:, 2 * CW:3 * CW] = b2[...] / d
    out[:, 3 * CW:4 * CW] = b3[...] / d


def _mean(blocks, deg8, n):
    bs = 2000
    grid = (pl.cdiv(n, bs),)
    return pl.pallas_call(
        _mean_body,
        grid=grid,
        in_specs=[pl.BlockSpec((bs, CW), lambda i: (i, 0)) for _ in range(CB)]
        + [pl.BlockSpec((bs, DEGW), lambda i: (i, 0))],
        out_specs=pl.BlockSpec((bs, D), lambda i: (i, 0)),
        out_shape=jax.ShapeDtypeStruct((n, D), jnp.float32),
    )(*blocks, deg8)


def kernel(feat_user, feat_item, edge_index):
    NU, _ = feat_user.shape
    NI, _ = feat_item.shape
    E = edge_index.shape[1]
    N = max(NU, NI)

    src = edge_index[0].astype(jnp.int32)
    dst = edge_index[1].astype(jnp.int32)
    EP = -(-E // K) * K
    if EP != E:
        pad_g = jnp.zeros((EP - E,), jnp.int32)
        pad_s = jnp.full((EP - E,), N, jnp.int32)
        gsrc = jnp.concatenate([src, pad_g])
        ssrc = jnp.concatenate([src, pad_s])
        gdst = jnp.concatenate([dst, pad_g])
        sdst = jnp.concatenate([dst, pad_s])
    else:
        gsrc, ssrc, gdst, sdst = src, src, dst, dst

    fu = [feat_user[:, cb * CW:(cb + 1) * CW] for cb in range(CB)]
    fi = [feat_item[:, cb * CW:(cb + 1) * CW] for cb in range(CB)]

    call = _sc_call(NU, NI, E)
    NPAD = -(-(N + 8) // (NS * 8)) * (NS * 8)
    RPT = NPAD // NS
    ones_h = jnp.ones((K, DEGW), jnp.float32)
    z32_h = jnp.zeros((RPT, CW), jnp.float32)
    z8_h = jnp.zeros((RPT, DEGW), jnp.float32)

    outs = call(*fu, *fi, gsrc, sdst, gdst, ssrc, ones_h, z32_h, z8_h)
    si_blocks = outs[0:4]
    su_blocks = outs[4:8]
    di8, du8 = outs[8], outs[9]

    h_item = _mean(si_blocks, di8, NI)
    h_user = _mean(su_blocks, du8, NU)
    return (h_user, h_item)


# speedup vs baseline: 4.9952x; 1.7927x over previous
import functools
import math

import jax
import jax.numpy as jnp
from jax import lax
from jax.experimental import pallas as pl
from jax.experimental.pallas import tpu as pltpu
from jax.experimental.pallas import tpu_sc as plsc

D = 128
CB = 4
CW = D // CB
K = 128
NS = 16
BS = 1600


def _npad(n):
    m = NS * 8 * BS // math.gcd(NS * 8, BS)
    return int(-(-(n + 8) // m) * m)


@functools.lru_cache(maxsize=None)
def _sc_call(NU, NI, E):
    N = max(NU, NI)
    NPAD = _npad(N)
    RPT = NPAD // NS
    NCH = -(-E // K)
    ITERS = -(-NCH // NS)
    mesh = plsc.VectorSubcoreMesh(core_axis_name="c", subcore_axis_name="s")
    f32 = jnp.float32
    last = ITERS - 1

    def body(fu_cat, fi_cat, gi_i, si_i, gi_u, si_u, ones_h, z32_h,
             outi, outu, degi, degu,
             acc,
             g0, g1, g2, s0, s1, s2, r0, r1, r2, ones_v,
             is0, is1, is2, gsem, ss0, ss1, ss2):
        c = lax.axis_index("c")
        s = lax.axis_index("s")
        gbuf = [g0, g1, g2]
        sbuf = [s0, s1, s2]
        rbuf = [r0, r1, r2]
        isem = [is0, is1, is2]
        ssem = [ss0, ss1, ss2]
        pltpu.sync_copy(ones_h, ones_v)

        def run_side(table, n_tab, gi, si, out, dego):
            def run_cb(cb, carry):
                off = cb * n_tab
                is_deg = cb == CB
                is_feat = cb != CB

                pltpu.sync_copy(z32_h.at[pl.ds(s * RPT, RPT)],
                                acc.at[pl.ds(s * RPT, RPT)])
                plsc.subcore_barrier()

                def fetch_idx(r, j):
                    base = (j * NS + s) * K
                    pltpu.async_copy(gi.at[pl.ds(base, K)], gbuf[r], isem[r])
                    pltpu.async_copy(si.at[pl.ds(base, K)], sbuf[r], isem[r])

                def gather_launch(r):
                    pltpu.make_async_copy(gi.at[pl.ds(0, K)], gbuf[r],
                                          isem[r]).wait()
                    pltpu.make_async_copy(si.at[pl.ds(0, K)], sbuf[r],
                                          isem[r]).wait()

                    @pl.when(is_feat)
                    def _():
                        for o in range(0, K, 16):
                            gbuf[r][pl.ds(o, 16)] = gbuf[r][pl.ds(o, 16)] + off
                        pltpu.async_copy(table.at[gbuf[r]], rbuf[r], gsem)

                def wait_gather(r):
                    @pl.when(is_feat)
                    def _():
                        pltpu.make_async_copy(table.at[gbuf[r]], rbuf[r],
                                              gsem).wait()

                def issue_scat(r):
                    @pl.when(is_feat)
                    def _():
                        pltpu.async_copy(rbuf[r], acc.at[sbuf[r]], ssem[r],
                                         add=True)

                    @pl.when(is_deg)
                    def _():
                        pltpu.async_copy(ones_v, acc.at[sbuf[r]], ssem[r],
                                         add=True)

                def wait_scat(r):
                    pltpu.make_async_copy(rbuf[r], acc.at[sbuf[r]],
                                          ssem[r]).wait()

                def step(j, r, first, has_next, has_fetch):
                    wait_gather(r)
                    if has_next:
                        gather_launch((r + 1) % 3)
                    issue_scat(r)
                    if not first:
                        wait_scat((r + 2) % 3)
                    if has_fetch:
                        fetch_idx((r + 2) % 3, j + 2)

                fetch_idx(0, 0)
                if last >= 1:
                    fetch_idx(1, 1)
                gather_launch(0)

                def flags(j):
                    return dict(first=(j == 0), has_next=(j + 1 <= last),
                                has_fetch=(j + 2 <= last))

                steady_lo, steady_hi = 3, last - 2
                n_steady = max(0, steady_hi - steady_lo + 1)
                groups, rem = divmod(n_steady, 3)

                for j in range(0, min(3, ITERS)):
                    step(j, j % 3, **flags(j))

                if groups > 0:
                    def grp(t, carry):
                        for js in range(3):
                            step(steady_lo + 3 * t + js, js, first=False,
                                 has_next=True, has_fetch=True)
                        return carry
                    lax.fori_loop(0, groups, grp, jnp.int32(0))

                for j in range(steady_lo + 3 * groups,
                               steady_lo + 3 * groups + rem):
                    step(j, j % 3, first=False, has_next=True, has_fetch=True)

                for j in range(max(3, last - 1), last + 1):
                    step(j, j % 3, **flags(j))

                wait_scat(last % 3)

                plsc.subcore_barrier()

                @pl.when(is_feat)
                def _():
                    pltpu.sync_copy(
                        acc.at[pl.ds(s * RPT, RPT)],
                        out.at[pl.ds(cb * NPAD + s * RPT, RPT)])

                @pl.when(is_deg)
                def _():
                    pltpu.sync_copy(acc.at[pl.ds(s * RPT, RPT)],
                                    dego.at[pl.ds(s * RPT, RPT)])

                plsc.subcore_barrier()
                return carry

            lax.fori_loop(0, CB + 1, run_cb, jnp.int32(0))

        @pl.when(c == 0)
        def _():
            run_side(fu_cat, NU, gi_i, si_i, outi, degi)

        @pl.when(c == 1)
        def _():
            run_side(fi_cat, NI, gi_u, si_u, outu, degu)

    out_type = [jax.ShapeDtypeStruct((CB * NPAD, CW), f32),
                jax.ShapeDtypeStruct((CB * NPAD, CW), f32),
                jax.ShapeDtypeStruct((NPAD, CW), f32),
                jax.ShapeDtypeStruct((NPAD, CW), f32)]
    scratch_types = (
        [pltpu.VMEM_SHARED((NPAD, CW), f32)]
        + [pltpu.VMEM((K,), jnp.int32) for _ in range(6)]
        + [pltpu.VMEM((K, CW), f32) for _ in range(3)]
        + [pltpu.VMEM((K, CW), f32)]
        + [pltpu.SemaphoreType.DMA for _ in range(7)]
    )
    return pl.kernel(body, out_type=out_type, mesh=mesh,
                     scratch_types=scratch_types,
                     compiler_params=pltpu.CompilerParams(
                         use_tc_tiling_on_sc=False))


def _mean_body(b0, b1, b2, b3, dg, out):
    d = jnp.maximum(dg[:, 0:1], 1.0)
    out[:, 0 * CW:1 * CW] = b0[...] / d
    out[:, 1 * CW:2 * CW] = b1[...] / d
    out[:, 2 * CW:3 * CW] = b2[...] / d
    out[:, 3 * CW:4 * CW] = b3[...] / d


def _mean(blocks_cat, deg, n, npad):
    nblk = npad // BS
    specs = [pl.BlockSpec((BS, CW), lambda i, _cb=cb: (_cb * nblk + i, 0))
             for cb in range(CB)]
    return pl.pallas_call(
        _mean_body,
        grid=(pl.cdiv(n, BS),),
        in_specs=specs + [pl.BlockSpec((BS, CW), lambda i: (i, 0))],
        out_specs=pl.BlockSpec((BS, D), lambda i: (i, 0)),
        out_shape=jax.ShapeDtypeStruct((n, D), jnp.float32),
    )(*([blocks_cat] * CB), deg)


def kernel(feat_user, feat_item, edge_index):
    NU, _ = feat_user.shape
    NI, _ = feat_item.shape
    E = edge_index.shape[1]
    N = max(NU, NI)
    NPAD = _npad(N)
    NCH = -(-E // K)
    ITERS = -(-NCH // NS)
    NCHP = ITERS * NS
    EP = NCHP * K

    src = edge_index[0].astype(jnp.int32)
    dst = edge_index[1].astype(jnp.int32)
    if EP != E:
        npd = EP - E
        pad_g = (jnp.arange(npd, dtype=jnp.int32) % N)
        pad_s = N + (jnp.arange(npd, dtype=jnp.int32) % (NPAD - N))
        gsrc = jnp.concatenate([src, pad_g])
        ssrc = jnp.concatenate([src, pad_s])
        gdst = jnp.concatenate([dst, pad_g])
        sdst = jnp.concatenate([dst, pad_s])
    else:
        gsrc, ssrc, gdst, sdst = src, src, dst, dst

    fu_cat = feat_user.reshape(NU, CB, CW).transpose(1, 0, 2).reshape(CB * NU, CW)
    fi_cat = feat_item.reshape(NI, CB, CW).transpose(1, 0, 2).reshape(CB * NI, CW)

    ones_h = jnp.ones((K, CW), jnp.float32)
    z32_h = jnp.zeros((NPAD, CW), jnp.float32)

    call = _sc_call(NU, NI, E)
    outi, outu, degi, degu = call(fu_cat, fi_cat, gsrc, sdst, gdst, ssrc,
                                  ones_h, z32_h)

    h_item = _mean(outi, degi, NI, NPAD)
    h_user = _mean(outu, degu, NU, NPAD)
    return (h_user, h_item)


# speedup vs baseline: 5.0681x; 1.0146x over previous
import functools
import math

import jax
import jax.numpy as jnp
from jax import lax
from jax.experimental import pallas as pl
from jax.experimental.pallas import tpu as pltpu
from jax.experimental.pallas import tpu_sc as plsc

D = 128
CB = 4
CW = D // CB
K = 128
NS = 16
BS = 1600


def _npad(n):
    m = NS * 8 * BS // math.gcd(NS * 8, BS)
    return int(-(-(n + 8) // m) * m)


@functools.lru_cache(maxsize=None)
def _sc_call(NU, NI, E):
    N = max(NU, NI)
    NPAD = _npad(N)
    RPT = NPAD // NS
    NCH = -(-E // K)
    ITERS = -(-NCH // NS)
    mesh = plsc.VectorSubcoreMesh(core_axis_name="c", subcore_axis_name="s")
    f32 = jnp.float32
    last = ITERS - 1

    def body(fu_cat, fi_cat, gi_i, si_i, gi_u, si_u, ones_h, z32_h,
             outi, outu, degi, degu,
             acc,
             g0, g1, g2, s0, s1, s2, r0, r1, r2, ones_v, iwait,
             is0, is1, is2, gsem, ss0, ss1, ss2):
        c = lax.axis_index("c")
        s = lax.axis_index("s")
        gbuf = [g0, g1, g2]
        sbuf = [s0, s1, s2]
        rbuf = [r0, r1, r2]
        isem = [is0, is1, is2]
        ssem = [ss0, ss1, ss2]
        pltpu.sync_copy(ones_h, ones_v)

        def run_side(table, n_tab, gi, si, out, dego):
            def run_cb(cb, carry):
                off = cb * n_tab
                is_deg = cb == CB
                is_feat = cb != CB

                pltpu.sync_copy(z32_h.at[pl.ds(s * RPT, RPT)],
                                acc.at[pl.ds(s * RPT, RPT)])
                plsc.subcore_barrier()

                def fetch_idx(r, j):
                    base = (j * NS + s) * K
                    pltpu.async_copy(gi.at[pl.ds(base, K)], gbuf[r], isem[r])
                    pltpu.async_copy(si.at[pl.ds(base, K)], sbuf[r], isem[r])

                def gather_launch(r):
                    pltpu.make_async_copy(gi.at[pl.ds(0, 2 * K)], iwait,
                                          isem[r]).wait()

                    @pl.when(is_feat)
                    def _():
                        for o in range(0, K, 16):
                            gbuf[r][pl.ds(o, 16)] = gbuf[r][pl.ds(o, 16)] + off
                        pltpu.async_copy(table.at[gbuf[r]], rbuf[r], gsem)

                def wait_gather(r):
                    @pl.when(is_feat)
                    def _():
                        pltpu.make_async_copy(table.at[gbuf[r]], rbuf[r],
                                              gsem).wait()

                def issue_scat(r):
                    @pl.when(is_feat)
                    def _():
                        pltpu.async_copy(rbuf[r], acc.at[sbuf[r]], ssem[r],
                                         add=True)

                    @pl.when(is_deg)
                    def _():
                        pltpu.async_copy(ones_v, acc.at[sbuf[r]], ssem[r],
                                         add=True)

                def wait_scat(r):
                    pltpu.make_async_copy(rbuf[r], acc.at[sbuf[r]],
                                          ssem[r]).wait()

                def step(j, r, first, has_next, has_fetch):
                    wait_gather(r)
                    if has_next:
                        gather_launch((r + 1) % 3)
                    issue_scat(r)
                    if not first:
                        wait_scat((r + 2) % 3)
                    if has_fetch:
                        fetch_idx((r + 2) % 3, j + 2)

                fetch_idx(0, 0)
                if last >= 1:
                    fetch_idx(1, 1)
                gather_launch(0)

                def flags(j):
                    return dict(first=(j == 0), has_next=(j + 1 <= last),
                                has_fetch=(j + 2 <= last))

                steady_lo, steady_hi = 3, last - 2
                n_steady = max(0, steady_hi - steady_lo + 1)
                groups, rem = divmod(n_steady, 3)

                for j in range(0, min(3, ITERS)):
                    step(j, j % 3, **flags(j))

                if groups > 0:
                    def grp(t, carry):
                        for js in range(3):
                            step(steady_lo + 3 * t + js, js, first=False,
                                 has_next=True, has_fetch=True)
                        return carry
                    lax.fori_loop(0, groups, grp, jnp.int32(0))

                for j in range(steady_lo + 3 * groups,
                               steady_lo + 3 * groups + rem):
                    step(j, j % 3, first=False, has_next=True, has_fetch=True)

                for j in range(max(3, last - 1), last + 1):
                    step(j, j % 3, **flags(j))

                wait_scat(last % 3)

                plsc.subcore_barrier()

                @pl.when(is_feat)
                def _():
                    pltpu.sync_copy(
                        acc.at[pl.ds(s * RPT, RPT)],
                        out.at[pl.ds(cb * NPAD + s * RPT, RPT)])

                @pl.when(is_deg)
                def _():
                    pltpu.sync_copy(acc.at[pl.ds(s * RPT, RPT)],
                                    dego.at[pl.ds(s * RPT, RPT)])

                plsc.subcore_barrier()
                return carry

            lax.fori_loop(0, CB + 1, run_cb, jnp.int32(0))

        @pl.when(c == 0)
        def _():
            run_side(fu_cat, NU, gi_i, si_i, outi, degi)

        @pl.when(c == 1)
        def _():
            run_side(fi_cat, NI, gi_u, si_u, outu, degu)

    out_type = [jax.ShapeDtypeStruct((CB * NPAD, CW), f32),
                jax.ShapeDtypeStruct((CB * NPAD, CW), f32),
                jax.ShapeDtypeStruct((NPAD, CW), f32),
                jax.ShapeDtypeStruct((NPAD, CW), f32)]
    scratch_types = (
        [pltpu.VMEM_SHARED((NPAD, CW), f32)]
        + [pltpu.VMEM((K,), jnp.int32) for _ in range(6)]
        + [pltpu.VMEM((K, CW), f32) for _ in range(3)]
        + [pltpu.VMEM((K, CW), f32)]
        + [pltpu.VMEM((2 * K,), jnp.int32)]
        + [pltpu.SemaphoreType.DMA for _ in range(7)]
    )
    return pl.kernel(body, out_type=out_type, mesh=mesh,
                     scratch_types=scratch_types,
                     compiler_params=pltpu.CompilerParams(
                         use_tc_tiling_on_sc=False))


def _mean_body(b0, b1, b2, b3, dg, out):
    d = jnp.maximum(dg[:, 0:1], 1.0)
    out[:, 0 * CW:1 * CW] = b0[...] / d
    out[:, 1 * CW:2 * CW] = b1[...] / d
    out[:, 2 * CW:3 * CW] = b2[...] / d
    out[:, 3 * CW:4 * CW] = b3[...] / d


def _mean2_body(a0, a1, a2, a3, da, b0, b1, b2, b3, db, oa, ob):
    _mean_body(a0, a1, a2, a3, da, oa)
    _mean_body(b0, b1, b2, b3, db, ob)


def _mean_pair(bi, di, bu, du, n, npad):
    nblk = npad // BS
    specs = [pl.BlockSpec((BS, CW), lambda i, _cb=cb: (_cb * nblk + i, 0))
             for cb in range(CB)] + [pl.BlockSpec((BS, CW), lambda i: (i, 0))]
    ospec = pl.BlockSpec((BS, D), lambda i: (i, 0))
    oshape = jax.ShapeDtypeStruct((n, D), jnp.float32)
    return pl.pallas_call(
        _mean2_body,
        grid=(pl.cdiv(n, BS),),
        in_specs=specs + specs,
        out_specs=[ospec, ospec],
        out_shape=[oshape, oshape],
    )(*([bi] * CB), di, *([bu] * CB), du)


def _mean(blocks_cat, deg, n, npad):
    nblk = npad // BS
    specs = [pl.BlockSpec((BS, CW), lambda i, _cb=cb: (_cb * nblk + i, 0))
             for cb in range(CB)]
    return pl.pallas_call(
        _mean_body,
        grid=(pl.cdiv(n, BS),),
        in_specs=specs + [pl.BlockSpec((BS, CW), lambda i: (i, 0))],
        out_specs=pl.BlockSpec((BS, D), lambda i: (i, 0)),
        out_shape=jax.ShapeDtypeStruct((n, D), jnp.float32),
    )(*([blocks_cat] * CB), deg)


def kernel(feat_user, feat_item, edge_index):
    NU, _ = feat_user.shape
    NI, _ = feat_item.shape
    E = edge_index.shape[1]
    N = max(NU, NI)
    NPAD = _npad(N)
    NCH = -(-E // K)
    ITERS = -(-NCH // NS)
    NCHP = ITERS * NS
    EP = NCHP * K

    src = edge_index[0].astype(jnp.int32)
    dst = edge_index[1].astype(jnp.int32)
    if EP != E:
        npd = EP - E
        pad_g = (jnp.arange(npd, dtype=jnp.int32) % N)
        pad_s = N + (jnp.arange(npd, dtype=jnp.int32) % (NPAD - N))
        gsrc = jnp.concatenate([src, pad_g])
        ssrc = jnp.concatenate([src, pad_s])
        gdst = jnp.concatenate([dst, pad_g])
        sdst = jnp.concatenate([dst, pad_s])
    else:
        gsrc, ssrc, gdst, sdst = src, src, dst, dst

    fu_cat = feat_user.reshape(NU, CB, CW).transpose(1, 0, 2).reshape(CB * NU, CW)
    fi_cat = feat_item.reshape(NI, CB, CW).transpose(1, 0, 2).reshape(CB * NI, CW)

    ones_h = jnp.ones((K, CW), jnp.float32)
    z32_h = jnp.zeros((NPAD, CW), jnp.float32)

    call = _sc_call(NU, NI, E)
    outi, outu, degi, degu = call(fu_cat, fi_cat, gsrc, sdst, gdst, ssrc,
                                  ones_h, z32_h)

    if NU == NI:
        h_item, h_user = _mean_pair(outi, degi, outu, degu, NU, NPAD)
    else:
        h_item = _mean(outi, degi, NI, NPAD)
        h_user = _mean(outu, degu, NU, NPAD)
    return (h_user, h_item)
